# baseline (speedup 1.0000x reference)
import functools

import jax
import jax.numpy as jnp
from jax import lax
from jax.experimental import pallas as pl
from jax.experimental.pallas import tpu as pltpu
from jax.experimental.pallas import tpu_sc as plsc

_D = 2048
_SEQ = 4096
_PREC = 10
_C = 16
_NW = 32
_LANES = 16


def _sc_body(x_hbm, emb_hbm, out_hbm, emb_v, buf0, buf1, keep, gs0, gs1,
             ss0, ss1):
    nrows = x_hbm.shape[0]
    rows_w = nrows // _NW
    nchunk = rows_w // _C
    wid = lax.axis_index("s") * 2 + lax.axis_index("c")
    base = wid * rows_w
    pltpu.sync_copy(emb_hbm, emb_v)
    bufs = (buf0, buf1)
    gsems = (gs0, gs1)
    ssems = (ss0, ss1)

    def gather(k, b):
        pltpu.async_copy(x_hbm.at[pl.ds(base + k * _C, _C)], bufs[b], gsems[b])

    def gwait(k, b):
        pltpu.make_async_copy(
            x_hbm.at[pl.ds(base + k * _C, _C)], bufs[b], gsems[b]).wait()

    def scatter(k, b):
        pltpu.async_copy(bufs[b], out_hbm.at[pl.ds(base + k * _C, _C)], ssems[b])

    def swait(k, b):
        pltpu.make_async_copy(
            bufs[b], out_hbm.at[pl.ds(base + k * _C, _C)], ssems[b]).wait()

    for b in range(2):
        gather(b, b)

    def compute(k, b):
        row0 = base + k * _C

        @plsc.parallel_loop(0, _C, unroll=2, carry=jnp.int32(0))
        def acc(r, c):
            digit = lax.rem((row0 + r) & (_SEQ - 1), _PREC)
            for j in range(_D // _LANES):
                sl = pl.ds(j * _LANES, _LANES)
                plsc.addupdate(bufs[b].at[r, sl], emb_v[digit, sl])
            return c

        keep[0] = acc

    def pair_body(p, carry):
        for b in range(2):
            k = 2 * p + b
            gwait(k, b)
            compute(k, b)
            scatter(k, b)

            @pl.when(k + 2 < nchunk)
            def _():
                swait(k, b)
                gather(k + 2, b)

        return carry

    lax.fori_loop(0, nchunk // 2, pair_body, 0)
    for b in range(2):
        swait(nchunk - 2 + b, b)


@functools.partial(
    pl.kernel,
    out_type=jax.ShapeDtypeStruct((4 * 4096, _D), jnp.float32),
    mesh=plsc.VectorSubcoreMesh(
        core_axis_name="c", subcore_axis_name="s",
        num_cores=2, num_subcores=16),
    scratch_types=[
        pltpu.VMEM((_PREC, _D), jnp.float32),
        pltpu.VMEM((_C, _D), jnp.float32),
        pltpu.VMEM((_C, _D), jnp.float32),
        pltpu.SMEM((1,), jnp.int32),
        pltpu.SemaphoreType.DMA,
        pltpu.SemaphoreType.DMA,
        pltpu.SemaphoreType.DMA,
        pltpu.SemaphoreType.DMA,
    ],
)
def _sc_kernel(x_hbm, emb_hbm, out_hbm, emb_v, buf0, buf1, keep, gs0, gs1,
               ss0, ss1):
    _sc_body(x_hbm, emb_hbm, out_hbm, emb_v, buf0, buf1, keep, gs0, gs1,
             ss0, ss1)


def kernel(x, embedding):
    b, s, d = x.shape
    out = _sc_kernel(x.reshape(b * s, d), embedding)
    return out.reshape(b, s, d)


# speedup vs baseline: 1.0753x; 1.0753x over previous
import functools

import jax
import jax.numpy as jnp
from jax import lax
from jax.experimental import pallas as pl
from jax.experimental.pallas import tpu as pltpu
from jax.experimental.pallas import tpu_sc as plsc

_D = 2048
_SEQ = 4096
_PREC = 10
_C = 8
_NBUF = 4
_NW = 32
_LANES = 16


def _sc_body(x_hbm, emb_hbm, out_hbm, emb_v, bufs, keep, gsems, ssems):
    nrows = x_hbm.shape[0]
    rows_w = nrows // _NW
    nchunk = rows_w // _C
    wid = lax.axis_index("s") * 2 + lax.axis_index("c")
    base = wid * rows_w
    pltpu.sync_copy(emb_hbm, emb_v)

    def gather(k, b):
        pltpu.async_copy(x_hbm.at[pl.ds(base + k * _C, _C)], bufs[b], gsems[b])

    def gwait(k, b):
        pltpu.make_async_copy(
            x_hbm.at[pl.ds(base + k * _C, _C)], bufs[b], gsems[b]).wait()

    def scatter(k, b):
        pltpu.async_copy(bufs[b], out_hbm.at[pl.ds(base + k * _C, _C)], ssems[b])

    def swait(k, b):
        pltpu.make_async_copy(
            bufs[b], out_hbm.at[pl.ds(base + k * _C, _C)], ssems[b]).wait()

    def compute(k, b):
        row0 = base + k * _C

        @plsc.parallel_loop(0, _C, unroll=4, carry=jnp.int32(0))
        def acc(r, c):
            digit = lax.rem((row0 + r) & (_SEQ - 1), _PREC)
            for j in range(_D // _LANES):
                sl = pl.ds(j * _LANES, _LANES)
                plsc.addupdate(bufs[b].at[r, sl], emb_v[digit, sl])
            return c

        keep[0] = acc

    for k in range(2):
        gather(k, k % _NBUF)

    def quad_body(q, carry):
        for i in range(_NBUF):
            k = _NBUF * q + i
            b = i
            gwait(k, b)
            compute(k, b)
            scatter(k, b)
            b2 = (i + 2) % _NBUF

            @pl.when(k >= 2)
            def _():
                swait(k - 2, b2)

            @pl.when(k + 2 < nchunk)
            def _():
                gather(k + 2, b2)

        return carry

    lax.fori_loop(0, nchunk // _NBUF, quad_body, 0)
    for k in range(nchunk - 2, nchunk):
        swait(k, k % _NBUF)


@functools.partial(
    pl.kernel,
    out_type=jax.ShapeDtypeStruct((4 * 4096, _D), jnp.float32),
    mesh=plsc.VectorSubcoreMesh(
        core_axis_name="c", subcore_axis_name="s",
        num_cores=2, num_subcores=16),
    scratch_types=(
        [pltpu.VMEM((_PREC, _D), jnp.float32)]
        + [pltpu.VMEM((_C, _D), jnp.float32) for _ in range(_NBUF)]
        + [pltpu.SMEM((1,), jnp.int32)]
        + [pltpu.SemaphoreType.DMA for _ in range(2 * _NBUF)]
    ),
)
def _sc_kernel(x_hbm, emb_hbm, out_hbm, emb_v, b0, b1, b2, b3, keep,
               g0, g1, g2, g3, s0, s1, s2, s3):
    _sc_body(x_hbm, emb_hbm, out_hbm, emb_v, (b0, b1, b2, b3), keep,
             (g0, g1, g2, g3), (s0, s1, s2, s3))


def kernel(x, embedding):
    b, s, d = x.shape
    out = _sc_kernel(x.reshape(b * s, d), embedding)
    return out.reshape(b, s, d)


# speedup vs baseline: 2.7199x; 2.5293x over previous
import jax
import jax.numpy as jnp
from jax.experimental import pallas as pl
from jax.experimental.pallas import tpu as pltpu

_PREC = 10
_SBLK = 1024


def _digit_add_kernel(x_ref, emb_ref, o_ref, emb_blk):
    j = pl.program_id(0)
    b = pl.program_id(1)

    @pl.when(b == 0)
    def _():
        base = j * _SBLK
        rows = (base + jax.lax.broadcasted_iota(
            jnp.int32, (_SBLK, _PREC), 0)) % _PREC
        cols = jax.lax.broadcasted_iota(jnp.int32, (_SBLK, _PREC), 1)
        onehot = (rows == cols).astype(jnp.float32)
        emb_blk[...] = jnp.dot(
            onehot, emb_ref[...], preferred_element_type=jnp.float32)

    o_ref[...] = x_ref[...] + emb_blk[...][None, :, :]


def kernel(x, embedding):
    b, s, d = x.shape
    grid = (s // _SBLK, b)
    return pl.pallas_call(
        _digit_add_kernel,
        grid=grid,
        in_specs=[
            pl.BlockSpec((1, _SBLK, d), lambda j, i: (i, j, 0)),
            pl.BlockSpec((_PREC, d), lambda j, i: (0, 0)),
        ],
        out_specs=pl.BlockSpec((1, _SBLK, d), lambda j, i: (i, j, 0)),
        out_shape=jax.ShapeDtypeStruct(x.shape, x.dtype),
        scratch_shapes=[pltpu.VMEM((_SBLK, d), jnp.float32)],
        compiler_params=pltpu.CompilerParams(
            dimension_semantics=("parallel", "arbitrary"),
        ),
    )(x, embedding)


# speedup vs baseline: 2.7319x; 1.0044x over previous
import jax
import jax.numpy as jnp
from jax.experimental import pallas as pl
from jax.experimental.pallas import tpu as pltpu

_PREC = 10
_SBLK = 1024


def _digit_add_kernel(x_ref, emb_ref, o_ref):
    j = pl.program_id(1)
    base = j * _SBLK
    rows = (base + jax.lax.broadcasted_iota(jnp.int32, (_SBLK, _PREC), 0)) % _PREC
    cols = jax.lax.broadcasted_iota(jnp.int32, (_SBLK, _PREC), 1)
    onehot = (rows == cols).astype(jnp.float32)
    emb_blk = jnp.dot(onehot, emb_ref[...], preferred_element_type=jnp.float32)
    o_ref[...] = x_ref[...] + emb_blk[None, :, :]


def kernel(x, embedding):
    b, s, d = x.shape
    grid = (b, s // _SBLK)
    return pl.pallas_call(
        _digit_add_kernel,
        grid=grid,
        in_specs=[
            pl.BlockSpec((1, _SBLK, d), lambda i, j: (i, j, 0)),
            pl.BlockSpec((_PREC, d), lambda i, j: (0, 0)),
        ],
        out_specs=pl.BlockSpec((1, _SBLK, d), lambda i, j: (i, j, 0)),
        out_shape=jax.ShapeDtypeStruct(x.shape, x.dtype),
        compiler_params=pltpu.CompilerParams(
            dimension_semantics=("parallel", "parallel"),
        ),
    )(x, embedding)
